# Initial kernel scaffold; baseline (speedup 1.0000x reference)
import jax
import jax.numpy as jnp
from jax.experimental import pallas as pl


def kernel(x, edge_index, Wg1, bg1, Wg2, bg2, Wl1, bl1, Wl2, bl2):
    raise NotImplementedError("write your pallas kernel here")


# speedup vs baseline: 5.2736x; 5.2736x over previous
import functools

import jax
import jax.numpy as jnp
from jax import lax
from jax.experimental import pallas as pl
from jax.experimental.pallas import tpu as pltpu
from jax.experimental.pallas import tpu_sc as plsc

N_NODES = 10000
N_EDGES = 320000
D_FEAT = 128
H_DIM = 512
EMB_DIM = 128

DP = 144
NC, NS = 2, 16
NW = NC * NS
EPW = N_EDGES // NW
CHUNK = 80
NCHUNK = EPW // CHUNK
RPT = 632
N_PAD = RPT * NS


def _sc_agg_body(xe_hbm, src_hbm, dst_hbm, zeros_hbm, agg_out,
                 src_v, dst_v, rows_v, acc_sh, sem):
    cid = lax.axis_index("c")
    sid = lax.axis_index("s")
    wid = sid * NC + cid
    r0 = sid * RPT
    pltpu.sync_copy(zeros_hbm, acc_sh.at[pl.ds(r0, RPT)])
    plsc.subcore_barrier()
    base = wid * EPW

    def body(i, _):
        off = base + i * CHUNK
        pltpu.sync_copy(src_hbm.at[pl.ds(off, CHUNK)], src_v)
        pltpu.sync_copy(dst_hbm.at[pl.ds(off, CHUNK)], dst_v)
        pltpu.async_copy(xe_hbm.at[src_v], rows_v, sem).wait()
        pltpu.sync_copy(rows_v, acc_sh.at[dst_v], add=True)
        return 0

    lax.fori_loop(0, NCHUNK, body, 0)
    plsc.subcore_barrier()
    pltpu.sync_copy(acc_sh.at[pl.ds(r0, RPT)],
                    agg_out.at[cid, pl.ds(r0, RPT)])


_sc_agg = functools.partial(
    pl.kernel,
    out_type=jax.ShapeDtypeStruct((NC, N_PAD, DP), jnp.float32),
    mesh=plsc.VectorSubcoreMesh(core_axis_name="c", subcore_axis_name="s"),
    compiler_params=pltpu.CompilerParams(use_tc_tiling_on_sc=False),
    scratch_types=[
        pltpu.VMEM((CHUNK,), jnp.int32),
        pltpu.VMEM((CHUNK,), jnp.int32),
        pltpu.VMEM((CHUNK, DP), jnp.float32),
        pltpu.VMEM_SHARED((N_PAD, DP), jnp.float32),
        pltpu.SemaphoreType.DMA,
    ],
)(_sc_agg_body)


BR = 1000


def _tc_body(x_ref, agg_ref, wg1a_ref, wg1b_ref, bg1_ref, wg2_ref,
             bg2_ref, wl1_ref, bl1_ref, wl2_ref, bl2_ref, outg_ref, outl_ref):
    xb = x_ref[...]
    a = agg_ref[0] + agg_ref[1]
    deg = a[:, D_FEAT:D_FEAT + 1]
    mean = a[:, :D_FEAT] / jnp.maximum(deg, 1.0)
    h = jnp.maximum(
        jnp.dot(xb, wg1a_ref[...], preferred_element_type=jnp.float32)
        + jnp.dot(mean, wg1b_ref[...], preferred_element_type=jnp.float32)
        + bg1_ref[...], 0.0)
    outg_ref[...] = jnp.dot(h, wg2_ref[...],
                            preferred_element_type=jnp.float32) + bg2_ref[...]
    hl = jnp.maximum(
        jnp.dot(xb, wl1_ref[...], preferred_element_type=jnp.float32)
        + bl1_ref[...], 0.0)
    outl_ref[...] = jnp.dot(hl, wl2_ref[...],
                            preferred_element_type=jnp.float32) + bl2_ref[...]


def _full(shape):
    return pl.BlockSpec(shape, lambda i: tuple(0 for _ in shape))


_tc_mlp = pl.pallas_call(
    _tc_body,
    grid=(N_NODES // BR,),
    in_specs=[
        pl.BlockSpec((BR, D_FEAT), lambda i: (i, 0)),
        pl.BlockSpec((NC, BR, DP), lambda i: (0, i, 0)),
        _full((D_FEAT, H_DIM)),
        _full((D_FEAT, H_DIM)),
        _full((1, H_DIM)),
        _full((H_DIM, EMB_DIM)),
        _full((1, EMB_DIM)),
        _full((D_FEAT, H_DIM)),
        _full((1, H_DIM)),
        _full((H_DIM, EMB_DIM)),
        _full((1, EMB_DIM)),
    ],
    out_specs=[
        pl.BlockSpec((BR, EMB_DIM), lambda i: (i, 0)),
        pl.BlockSpec((BR, EMB_DIM), lambda i: (i, 0)),
    ],
    out_shape=[
        jax.ShapeDtypeStruct((N_NODES, EMB_DIM), jnp.float32),
        jax.ShapeDtypeStruct((N_NODES, EMB_DIM), jnp.float32),
    ],
)


def kernel(x, edge_index, Wg1, bg1, Wg2, bg2, Wl1, bl1, Wl2, bl2):
    xe = jnp.concatenate(
        [x, jnp.ones((N_NODES, 1), jnp.float32),
         jnp.zeros((N_NODES, DP - D_FEAT - 1), jnp.float32)], axis=1)
    src = edge_index[0]
    dst = edge_index[1]
    zeros = jnp.zeros((RPT, DP), jnp.float32)
    agg = _sc_agg(xe, src, dst, zeros)
    out_g, out_l = _tc_mlp(x, agg,
                           Wg1[:D_FEAT], Wg1[D_FEAT:], bg1.reshape(1, H_DIM),
                           Wg2, bg2.reshape(1, EMB_DIM),
                           Wl1, bl1.reshape(1, H_DIM),
                           Wl2, bl2.reshape(1, EMB_DIM))
    return (out_g, out_l)


# speedup vs baseline: 9.5781x; 1.8162x over previous
import functools

import jax
import jax.numpy as jnp
from jax import lax
from jax.experimental import pallas as pl
from jax.experimental.pallas import tpu as pltpu
from jax.experimental.pallas import tpu_sc as plsc

N_NODES = 10000
N_EDGES = 320000
D_FEAT = 128
H_DIM = 512
EMB_DIM = 128

DP = 144
NC, NS = 2, 16
NW = NC * NS
EPW = N_EDGES // NW
CHUNK = 80
NCHUNK = EPW // CHUNK
G = 25
NGROUP = NCHUNK // G
RPT = 632
N_PAD = RPT * NS


def _sc_agg_body(xe_hbm, srcr_hbm, dstr_hbm, zeros_hbm, agg_out,
                 src_i, dst_i, buf_a, buf_b, acc_sh, sem_a, sem_b):
    cid = lax.axis_index("c")
    sid = lax.axis_index("s")
    wid = sid * NC + cid
    r0 = sid * RPT
    pltpu.sync_copy(zeros_hbm, acc_sh.at[pl.ds(r0, RPT)])
    plsc.subcore_barrier()

    def group(g, _):
        pltpu.sync_copy(srcr_hbm.at[wid, pl.ds(g * G, G)], src_i)
        pltpu.sync_copy(dstr_hbm.at[wid, pl.ds(g * G, G)], dst_i)
        pltpu.async_copy(xe_hbm.at[src_i.at[0]], buf_a, sem_a)

        def body(j, _):
            c0 = 2 * j
            pltpu.async_copy(xe_hbm.at[src_i.at[c0 + 1]], buf_b, sem_b)
            pltpu.make_async_copy(xe_hbm.at[src_i.at[c0]], buf_a, sem_a).wait()
            pltpu.sync_copy(buf_a, acc_sh.at[dst_i.at[c0]], add=True)
            pltpu.async_copy(xe_hbm.at[src_i.at[c0 + 2]], buf_a, sem_a)
            pltpu.make_async_copy(xe_hbm.at[src_i.at[c0 + 1]], buf_b,
                                  sem_b).wait()
            pltpu.sync_copy(buf_b, acc_sh.at[dst_i.at[c0 + 1]], add=True)
            return 0

        lax.fori_loop(0, (G - 1) // 2, body, 0)
        pltpu.make_async_copy(xe_hbm.at[src_i.at[G - 1]], buf_a, sem_a).wait()
        pltpu.sync_copy(buf_a, acc_sh.at[dst_i.at[G - 1]], add=True)
        return 0

    lax.fori_loop(0, NGROUP, group, 0)
    plsc.subcore_barrier()
    pltpu.sync_copy(acc_sh.at[pl.ds(r0, RPT)],
                    agg_out.at[cid, pl.ds(r0, RPT)])


_sc_agg = functools.partial(
    pl.kernel,
    out_type=jax.ShapeDtypeStruct((NC, N_PAD, DP), jnp.float32),
    mesh=plsc.VectorSubcoreMesh(core_axis_name="c", subcore_axis_name="s"),
    compiler_params=pltpu.CompilerParams(use_tc_tiling_on_sc=False),
    scratch_types=[
        pltpu.VMEM((G, CHUNK), jnp.int32),
        pltpu.VMEM((G, CHUNK), jnp.int32),
        pltpu.VMEM((CHUNK, DP), jnp.float32),
        pltpu.VMEM((CHUNK, DP), jnp.float32),
        pltpu.VMEM_SHARED((N_PAD, DP), jnp.float32),
        pltpu.SemaphoreType.DMA,
        pltpu.SemaphoreType.DMA,
    ],
)(_sc_agg_body)


BR = 1000


def _tc_body(x_ref, agg_ref, wg1a_ref, wg1b_ref, bg1_ref, wg2_ref,
             bg2_ref, wl1_ref, bl1_ref, wl2_ref, bl2_ref, outg_ref, outl_ref):
    xb = x_ref[...]
    a = agg_ref[0] + agg_ref[1]
    deg = a[:, D_FEAT:D_FEAT + 1]
    mean = a[:, :D_FEAT] / jnp.maximum(deg, 1.0)
    h = jnp.maximum(
        jnp.dot(xb, wg1a_ref[...], preferred_element_type=jnp.float32)
        + jnp.dot(mean, wg1b_ref[...], preferred_element_type=jnp.float32)
        + bg1_ref[...], 0.0)
    outg_ref[...] = jnp.dot(h, wg2_ref[...],
                            preferred_element_type=jnp.float32) + bg2_ref[...]
    hl = jnp.maximum(
        jnp.dot(xb, wl1_ref[...], preferred_element_type=jnp.float32)
        + bl1_ref[...], 0.0)
    outl_ref[...] = jnp.dot(hl, wl2_ref[...],
                            preferred_element_type=jnp.float32) + bl2_ref[...]


def _full(shape):
    return pl.BlockSpec(shape, lambda i: tuple(0 for _ in shape))


_tc_mlp = pl.pallas_call(
    _tc_body,
    grid=(N_NODES // BR,),
    in_specs=[
        pl.BlockSpec((BR, D_FEAT), lambda i: (i, 0)),
        pl.BlockSpec((NC, BR, DP), lambda i: (0, i, 0)),
        _full((D_FEAT, H_DIM)),
        _full((D_FEAT, H_DIM)),
        _full((1, H_DIM)),
        _full((H_DIM, EMB_DIM)),
        _full((1, EMB_DIM)),
        _full((D_FEAT, H_DIM)),
        _full((1, H_DIM)),
        _full((H_DIM, EMB_DIM)),
        _full((1, EMB_DIM)),
    ],
    out_specs=[
        pl.BlockSpec((BR, EMB_DIM), lambda i: (i, 0)),
        pl.BlockSpec((BR, EMB_DIM), lambda i: (i, 0)),
    ],
    out_shape=[
        jax.ShapeDtypeStruct((N_NODES, EMB_DIM), jnp.float32),
        jax.ShapeDtypeStruct((N_NODES, EMB_DIM), jnp.float32),
    ],
)


def kernel(x, edge_index, Wg1, bg1, Wg2, bg2, Wl1, bl1, Wl2, bl2):
    xe = jnp.concatenate(
        [x, jnp.ones((N_NODES, 1), jnp.float32),
         jnp.zeros((N_NODES, DP - D_FEAT - 1), jnp.float32)], axis=1)
    srcr = edge_index[0].reshape(NW, NCHUNK, CHUNK)
    dstr = edge_index[1].reshape(NW, NCHUNK, CHUNK)
    zeros = jnp.zeros((RPT, DP), jnp.float32)
    agg = _sc_agg(xe, srcr, dstr, zeros)
    out_g, out_l = _tc_mlp(x, agg,
                           Wg1[:D_FEAT], Wg1[D_FEAT:], bg1.reshape(1, H_DIM),
                           Wg2, bg2.reshape(1, EMB_DIM),
                           Wl1, bl1.reshape(1, H_DIM),
                           Wl2, bl2.reshape(1, EMB_DIM))
    return (out_g, out_l)


# speedup vs baseline: 10.3358x; 1.0791x over previous
import functools

import jax
import jax.numpy as jnp
from jax import lax
from jax.experimental import pallas as pl
from jax.experimental.pallas import tpu as pltpu
from jax.experimental.pallas import tpu_sc as plsc

N_NODES = 10000
N_EDGES = 320000
D_FEAT = 128
H_DIM = 512
EMB_DIM = 128

DP = 144
NC, NS = 2, 16
NW = NC * NS
EPW = N_EDGES // NW
CHUNK = 80
NCHUNK = EPW // CHUNK
G = 25
NGROUP = NCHUNK // G
RPT = 632
N_PAD = RPT * NS


def _sc_agg_body(xe_hbm, srcr_hbm, dstr_hbm, zeros_hbm, agg_out,
                 src_i, dst_i, bufs, acc_sh, gsems, ssems):
    cid = lax.axis_index("c")
    sid = lax.axis_index("s")
    wid = sid * NC + cid
    r0 = sid * RPT
    pltpu.sync_copy(zeros_hbm, acc_sh.at[pl.ds(r0, RPT)])
    plsc.subcore_barrier()

    def gather(c, bi):
        pltpu.async_copy(xe_hbm.at[src_i.at[c]], bufs[bi], gsems[bi])

    def gather_wait(c, bi):
        pltpu.make_async_copy(xe_hbm.at[src_i.at[c]], bufs[bi],
                              gsems[bi]).wait()

    def scat(c, bi):
        pltpu.async_copy(bufs[bi], acc_sh.at[dst_i.at[c]], ssems[bi],
                         add=True)

    def scat_wait(c, bi):
        pltpu.make_async_copy(bufs[bi], acc_sh.at[dst_i.at[c]],
                              ssems[bi]).wait()

    def group(g, _):
        pltpu.sync_copy(srcr_hbm.at[wid, pl.ds(g * G, G)], src_i)
        pltpu.sync_copy(dstr_hbm.at[wid, pl.ds(g * G, G)], dst_i)
        gather(0, 0)
        gather(1, 1)
        for c in range(G):
            bi = c % 3
            gather_wait(c, bi)
            scat(c, bi)
            if c + 2 < G:
                nbi = (c + 2) % 3
                if c >= 1:
                    scat_wait(c - 1, nbi)
                gather(c + 2, nbi)
        for c in range(G - 3, G):
            scat_wait(c, c % 3)
        return 0

    lax.fori_loop(0, NGROUP, group, 0)
    plsc.subcore_barrier()
    pltpu.sync_copy(acc_sh.at[pl.ds(r0, RPT)],
                    agg_out.at[cid, pl.ds(r0, RPT)])


_sc_agg = functools.partial(
    pl.kernel,
    out_type=jax.ShapeDtypeStruct((NC, N_PAD, DP), jnp.float32),
    mesh=plsc.VectorSubcoreMesh(core_axis_name="c", subcore_axis_name="s"),
    compiler_params=pltpu.CompilerParams(use_tc_tiling_on_sc=False),
    scratch_types=[
        pltpu.VMEM((G, CHUNK), jnp.int32),
        pltpu.VMEM((G, CHUNK), jnp.int32),
        [pltpu.VMEM((CHUNK, DP), jnp.float32)] * 3,
        pltpu.VMEM_SHARED((N_PAD, DP), jnp.float32),
        [pltpu.SemaphoreType.DMA] * 3,
        [pltpu.SemaphoreType.DMA] * 3,
    ],
)(_sc_agg_body)


BR = 1000


def _tc_body(x_ref, agg_ref, wg1a_ref, wg1b_ref, bg1_ref, wg2_ref,
             bg2_ref, wl1_ref, bl1_ref, wl2_ref, bl2_ref, outg_ref, outl_ref):
    xb = x_ref[...]
    a = agg_ref[0] + agg_ref[1]
    deg = a[:, D_FEAT:D_FEAT + 1]
    mean = a[:, :D_FEAT] / jnp.maximum(deg, 1.0)
    h = jnp.maximum(
        jnp.dot(xb, wg1a_ref[...], preferred_element_type=jnp.float32)
        + jnp.dot(mean, wg1b_ref[...], preferred_element_type=jnp.float32)
        + bg1_ref[...], 0.0)
    outg_ref[...] = jnp.dot(h, wg2_ref[...],
                            preferred_element_type=jnp.float32) + bg2_ref[...]
    hl = jnp.maximum(
        jnp.dot(xb, wl1_ref[...], preferred_element_type=jnp.float32)
        + bl1_ref[...], 0.0)
    outl_ref[...] = jnp.dot(hl, wl2_ref[...],
                            preferred_element_type=jnp.float32) + bl2_ref[...]


def _full(shape):
    return pl.BlockSpec(shape, lambda i: tuple(0 for _ in shape))


_tc_mlp = pl.pallas_call(
    _tc_body,
    grid=(N_NODES // BR,),
    in_specs=[
        pl.BlockSpec((BR, D_FEAT), lambda i: (i, 0)),
        pl.BlockSpec((NC, BR, DP), lambda i: (0, i, 0)),
        _full((D_FEAT, H_DIM)),
        _full((D_FEAT, H_DIM)),
        _full((1, H_DIM)),
        _full((H_DIM, EMB_DIM)),
        _full((1, EMB_DIM)),
        _full((D_FEAT, H_DIM)),
        _full((1, H_DIM)),
        _full((H_DIM, EMB_DIM)),
        _full((1, EMB_DIM)),
    ],
    out_specs=[
        pl.BlockSpec((BR, EMB_DIM), lambda i: (i, 0)),
        pl.BlockSpec((BR, EMB_DIM), lambda i: (i, 0)),
    ],
    out_shape=[
        jax.ShapeDtypeStruct((N_NODES, EMB_DIM), jnp.float32),
        jax.ShapeDtypeStruct((N_NODES, EMB_DIM), jnp.float32),
    ],
)


def kernel(x, edge_index, Wg1, bg1, Wg2, bg2, Wl1, bl1, Wl2, bl2):
    xe = jnp.concatenate(
        [x, jnp.ones((N_NODES, 1), jnp.float32),
         jnp.zeros((N_NODES, DP - D_FEAT - 1), jnp.float32)], axis=1)
    srcr = edge_index[0].reshape(NW, NCHUNK, CHUNK)
    dstr = edge_index[1].reshape(NW, NCHUNK, CHUNK)
    zeros = jnp.zeros((RPT, DP), jnp.float32)
    agg = _sc_agg(xe, srcr, dstr, zeros)
    out_g, out_l = _tc_mlp(x, agg,
                           Wg1[:D_FEAT], Wg1[D_FEAT:], bg1.reshape(1, H_DIM),
                           Wg2, bg2.reshape(1, EMB_DIM),
                           Wl1, bl1.reshape(1, H_DIM),
                           Wl2, bl2.reshape(1, EMB_DIM))
    return (out_g, out_l)


# speedup vs baseline: 12.1637x; 1.1769x over previous
import functools

import jax
import jax.numpy as jnp
from jax import lax
from jax.experimental import pallas as pl
from jax.experimental.pallas import tpu as pltpu
from jax.experimental.pallas import tpu_sc as plsc

N_NODES = 10000
N_EDGES = 320000
D_FEAT = 128
H_DIM = 512
EMB_DIM = 128

DW = 8
NC, NS = 2, 16
NW = NC * NS
EPW = N_EDGES // NW
CHUNK = 80
NCHUNK = EPW // CHUNK
G = 25
NGROUP = NCHUNK // G
RPT = 632
N_PAD = RPT * NS


def _sc_agg_body(x_hbm, srcr_hbm, dstr_hbm, zeros_hbm, ones_hbm,
                 agg_out, deg_out,
                 src_i, dst_i, ones_v, bufs, acc_sh, deg_sh,
                 gsems, ssems, dsem):
    cid = lax.axis_index("c")
    sid = lax.axis_index("s")
    wid = sid * NC + cid
    r0 = sid * RPT
    pltpu.sync_copy(ones_hbm, ones_v)
    pltpu.sync_copy(zeros_hbm, acc_sh.at[pl.ds(r0, RPT)])
    pltpu.sync_copy(zeros_hbm.at[pl.ds(0, RPT), pl.ds(0, DW)],
                    deg_sh.at[pl.ds(r0, RPT)])
    plsc.subcore_barrier()

    def gather(c, bi):
        pltpu.async_copy(x_hbm.at[src_i.at[c]], bufs[bi], gsems[bi])

    def gather_wait(c, bi):
        pltpu.make_async_copy(x_hbm.at[src_i.at[c]], bufs[bi],
                              gsems[bi]).wait()

    def scat(c, bi):
        pltpu.async_copy(bufs[bi], acc_sh.at[dst_i.at[c]], ssems[bi],
                         add=True)
        pltpu.async_copy(ones_v, deg_sh.at[dst_i.at[c]], dsem, add=True)

    def scat_wait(c, bi):
        pltpu.make_async_copy(bufs[bi], acc_sh.at[dst_i.at[c]],
                              ssems[bi]).wait()

    def deg_wait(c):
        pltpu.make_async_copy(ones_v, deg_sh.at[dst_i.at[c]], dsem).wait()

    def group(g, _):
        pltpu.sync_copy(srcr_hbm.at[wid, pl.ds(g * G, G)], src_i)
        pltpu.sync_copy(dstr_hbm.at[wid, pl.ds(g * G, G)], dst_i)
        gather(0, 0)
        gather(1, 1)
        for c in range(G):
            bi = c % 3
            gather_wait(c, bi)
            scat(c, bi)
            if c >= 1:
                deg_wait(c - 1)
            if c + 2 < G:
                nbi = (c + 2) % 3
                if c >= 1:
                    scat_wait(c - 1, nbi)
                gather(c + 2, nbi)
        for c in range(G - 3, G):
            scat_wait(c, c % 3)
        deg_wait(G - 1)
        return 0

    lax.fori_loop(0, NGROUP, group, 0)
    plsc.subcore_barrier()
    pltpu.sync_copy(acc_sh.at[pl.ds(r0, RPT)],
                    agg_out.at[cid, pl.ds(r0, RPT)])
    pltpu.sync_copy(deg_sh.at[pl.ds(r0, RPT)],
                    deg_out.at[cid, pl.ds(r0, RPT)])


_sc_agg = functools.partial(
    pl.kernel,
    out_type=(jax.ShapeDtypeStruct((NC, N_PAD, D_FEAT), jnp.float32),
              jax.ShapeDtypeStruct((NC, N_PAD, DW), jnp.float32)),
    mesh=plsc.VectorSubcoreMesh(core_axis_name="c", subcore_axis_name="s"),
    compiler_params=pltpu.CompilerParams(use_tc_tiling_on_sc=False),
    scratch_types=[
        pltpu.VMEM((G, CHUNK), jnp.int32),
        pltpu.VMEM((G, CHUNK), jnp.int32),
        pltpu.VMEM((CHUNK, DW), jnp.float32),
        [pltpu.VMEM((CHUNK, D_FEAT), jnp.float32)] * 3,
        pltpu.VMEM_SHARED((N_PAD, D_FEAT), jnp.float32),
        pltpu.VMEM_SHARED((N_PAD, DW), jnp.float32),
        [pltpu.SemaphoreType.DMA] * 3,
        [pltpu.SemaphoreType.DMA] * 3,
        pltpu.SemaphoreType.DMA,
    ],
)(_sc_agg_body)


BR = 1000


def _tc_body(x_ref, agg_ref, deg_ref, wg1a_ref, wg1b_ref, bg1_ref, wg2_ref,
             bg2_ref, wl1_ref, bl1_ref, wl2_ref, bl2_ref, outg_ref, outl_ref):
    xb = x_ref[...]
    a = agg_ref[0] + agg_ref[1]
    deg = deg_ref[0, :, 0:1] + deg_ref[1, :, 0:1]
    mean = a / jnp.maximum(deg, 1.0)
    h = jnp.maximum(
        jnp.dot(xb, wg1a_ref[...], preferred_element_type=jnp.float32)
        + jnp.dot(mean, wg1b_ref[...], preferred_element_type=jnp.float32)
        + bg1_ref[...], 0.0)
    outg_ref[...] = jnp.dot(h, wg2_ref[...],
                            preferred_element_type=jnp.float32) + bg2_ref[...]
    hl = jnp.maximum(
        jnp.dot(xb, wl1_ref[...], preferred_element_type=jnp.float32)
        + bl1_ref[...], 0.0)
    outl_ref[...] = jnp.dot(hl, wl2_ref[...],
                            preferred_element_type=jnp.float32) + bl2_ref[...]


def _full(shape):
    return pl.BlockSpec(shape, lambda i: tuple(0 for _ in shape))


_tc_mlp = pl.pallas_call(
    _tc_body,
    grid=(N_NODES // BR,),
    in_specs=[
        pl.BlockSpec((BR, D_FEAT), lambda i: (i, 0)),
        pl.BlockSpec((NC, BR, D_FEAT), lambda i: (0, i, 0)),
        pl.BlockSpec((NC, BR, DW), lambda i: (0, i, 0)),
        _full((D_FEAT, H_DIM)),
        _full((D_FEAT, H_DIM)),
        _full((1, H_DIM)),
        _full((H_DIM, EMB_DIM)),
        _full((1, EMB_DIM)),
        _full((D_FEAT, H_DIM)),
        _full((1, H_DIM)),
        _full((H_DIM, EMB_DIM)),
        _full((1, EMB_DIM)),
    ],
    out_specs=[
        pl.BlockSpec((BR, EMB_DIM), lambda i: (i, 0)),
        pl.BlockSpec((BR, EMB_DIM), lambda i: (i, 0)),
    ],
    out_shape=[
        jax.ShapeDtypeStruct((N_NODES, EMB_DIM), jnp.float32),
        jax.ShapeDtypeStruct((N_NODES, EMB_DIM), jnp.float32),
    ],
)


def kernel(x, edge_index, Wg1, bg1, Wg2, bg2, Wl1, bl1, Wl2, bl2):
    srcr = edge_index[0].reshape(NW, NCHUNK, CHUNK)
    dstr = edge_index[1].reshape(NW, NCHUNK, CHUNK)
    zeros = jnp.zeros((RPT, D_FEAT), jnp.float32)
    ones = jnp.ones((CHUNK, DW), jnp.float32)
    agg, deg = _sc_agg(x, srcr, dstr, zeros, ones)
    out_g, out_l = _tc_mlp(x, agg, deg,
                           Wg1[:D_FEAT], Wg1[D_FEAT:], bg1.reshape(1, H_DIM),
                           Wg2, bg2.reshape(1, EMB_DIM),
                           Wl1, bl1.reshape(1, H_DIM),
                           Wl2, bl2.reshape(1, EMB_DIM))
    return (out_g, out_l)


# speedup vs baseline: 13.0396x; 1.0720x over previous
import functools

import jax
import jax.numpy as jnp
from jax import lax
from jax.experimental import pallas as pl
from jax.experimental.pallas import tpu as pltpu
from jax.experimental.pallas import tpu_sc as plsc

N_NODES = 10000
N_EDGES = 320000
D_FEAT = 128
H_DIM = 512
EMB_DIM = 128

DW = 8
NC, NS = 2, 16
NW = NC * NS
EPW = N_EDGES // NW
CHUNK = 80
NCHUNK = EPW // CHUNK
G = 25
NGROUP = NCHUNK // G
RPT = 632
N_PAD = RPT * NS


def _sc_agg_body(x_hbm, er_hbm, zeros_hbm, ones_hbm,
                 agg_out, deg_out,
                 src_i, dst_i, ones_v, bufs, acc_sh, deg_sh,
                 gsems, ssems, dsem):
    cid = lax.axis_index("c")
    sid = lax.axis_index("s")
    wid = sid * NC + cid
    r0 = sid * RPT
    pltpu.sync_copy(ones_hbm, ones_v)
    pltpu.sync_copy(zeros_hbm, acc_sh.at[pl.ds(r0, RPT)])
    pltpu.sync_copy(zeros_hbm.at[pl.ds(0, RPT), pl.ds(0, DW)],
                    deg_sh.at[pl.ds(r0, RPT)])
    plsc.subcore_barrier()

    def gather(c, bi):
        pltpu.async_copy(x_hbm.at[src_i.at[c]], bufs[bi], gsems[bi])

    def gather_wait(c, bi):
        pltpu.make_async_copy(x_hbm.at[src_i.at[c]], bufs[bi],
                              gsems[bi]).wait()

    def scat(c, bi):
        pltpu.async_copy(bufs[bi], acc_sh.at[dst_i.at[c]], ssems[bi],
                         add=True)
        pltpu.async_copy(ones_v, deg_sh.at[dst_i.at[c]], dsem, add=True)

    def scat_wait(c, bi):
        pltpu.make_async_copy(bufs[bi], acc_sh.at[dst_i.at[c]],
                              ssems[bi]).wait()

    def deg_wait(c):
        pltpu.make_async_copy(ones_v, deg_sh.at[dst_i.at[c]], dsem).wait()

    def group(g, _):
        pltpu.sync_copy(er_hbm.at[0, wid, pl.ds(g * G, G)], src_i)
        pltpu.sync_copy(er_hbm.at[1, wid, pl.ds(g * G, G)], dst_i)
        gather(0, 0)
        gather(1, 1)
        for c in range(G):
            bi = c % 3
            gather_wait(c, bi)
            scat(c, bi)
            if c >= 1:
                deg_wait(c - 1)
            if c + 2 < G:
                nbi = (c + 2) % 3
                if c >= 1:
                    scat_wait(c - 1, nbi)
                gather(c + 2, nbi)
        for c in range(G - 3, G):
            scat_wait(c, c % 3)
        deg_wait(G - 1)
        return 0

    lax.fori_loop(0, NGROUP, group, 0)
    plsc.subcore_barrier()
    pltpu.sync_copy(acc_sh.at[pl.ds(r0, RPT)],
                    agg_out.at[cid, pl.ds(r0, RPT)])
    pltpu.sync_copy(deg_sh.at[pl.ds(r0, RPT)],
                    deg_out.at[cid, pl.ds(r0, RPT)])


_sc_agg = functools.partial(
    pl.kernel,
    out_type=(jax.ShapeDtypeStruct((NC, N_PAD, D_FEAT), jnp.float32),
              jax.ShapeDtypeStruct((NC, N_PAD, DW), jnp.float32)),
    mesh=plsc.VectorSubcoreMesh(core_axis_name="c", subcore_axis_name="s"),
    compiler_params=pltpu.CompilerParams(use_tc_tiling_on_sc=False),
    scratch_types=[
        pltpu.VMEM((G, CHUNK), jnp.int32),
        pltpu.VMEM((G, CHUNK), jnp.int32),
        pltpu.VMEM((CHUNK, DW), jnp.float32),
        [pltpu.VMEM((CHUNK, D_FEAT), jnp.float32)] * 3,
        pltpu.VMEM_SHARED((N_PAD, D_FEAT), jnp.float32),
        pltpu.VMEM_SHARED((N_PAD, DW), jnp.float32),
        [pltpu.SemaphoreType.DMA] * 3,
        [pltpu.SemaphoreType.DMA] * 3,
        pltpu.SemaphoreType.DMA,
    ],
)(_sc_agg_body)


BR = 1000


def _tc_global_body(x_ref, agg_ref, deg_ref, wg1a_ref, wg1b_ref, bg1_ref,
                    wg2_ref, bg2_ref, outg_ref):
    xb = x_ref[...]
    a = agg_ref[0] + agg_ref[1]
    deg = deg_ref[0, :, 0:1] + deg_ref[1, :, 0:1]
    mean = a / jnp.maximum(deg, 1.0)
    h = jnp.maximum(
        jnp.dot(xb, wg1a_ref[...], preferred_element_type=jnp.float32)
        + jnp.dot(mean, wg1b_ref[...], preferred_element_type=jnp.float32)
        + bg1_ref[...], 0.0)
    outg_ref[...] = jnp.dot(h, wg2_ref[...],
                            preferred_element_type=jnp.float32) + bg2_ref[...]


def _tc_local_body(x_ref, wl1_ref, bl1_ref, wl2_ref, bl2_ref, outl_ref):
    hl = jnp.maximum(
        jnp.dot(x_ref[...], wl1_ref[...], preferred_element_type=jnp.float32)
        + bl1_ref[...], 0.0)
    outl_ref[...] = jnp.dot(hl, wl2_ref[...],
                            preferred_element_type=jnp.float32) + bl2_ref[...]


def _full(shape):
    return pl.BlockSpec(shape, lambda i: tuple(0 for _ in shape))


_tc_global = pl.pallas_call(
    _tc_global_body,
    grid=(N_NODES // BR,),
    in_specs=[
        pl.BlockSpec((BR, D_FEAT), lambda i: (i, 0)),
        pl.BlockSpec((NC, BR, D_FEAT), lambda i: (0, i, 0)),
        pl.BlockSpec((NC, BR, DW), lambda i: (0, i, 0)),
        _full((D_FEAT, H_DIM)),
        _full((D_FEAT, H_DIM)),
        _full((1, H_DIM)),
        _full((H_DIM, EMB_DIM)),
        _full((1, EMB_DIM)),
    ],
    out_specs=pl.BlockSpec((BR, EMB_DIM), lambda i: (i, 0)),
    out_shape=jax.ShapeDtypeStruct((N_NODES, EMB_DIM), jnp.float32),
)

_tc_local = pl.pallas_call(
    _tc_local_body,
    grid=(N_NODES // BR,),
    in_specs=[
        pl.BlockSpec((BR, D_FEAT), lambda i: (i, 0)),
        _full((D_FEAT, H_DIM)),
        _full((1, H_DIM)),
        _full((H_DIM, EMB_DIM)),
        _full((1, EMB_DIM)),
    ],
    out_specs=pl.BlockSpec((BR, EMB_DIM), lambda i: (i, 0)),
    out_shape=jax.ShapeDtypeStruct((N_NODES, EMB_DIM), jnp.float32),
)


def kernel(x, edge_index, Wg1, bg1, Wg2, bg2, Wl1, bl1, Wl2, bl2):
    er = edge_index.reshape(2, NW, NCHUNK, CHUNK)
    zeros = jnp.zeros((RPT, D_FEAT), jnp.float32)
    ones = jnp.ones((CHUNK, DW), jnp.float32)
    agg, deg = _sc_agg(x, er, zeros, ones)
    out_l = _tc_local(x, Wl1, bl1.reshape(1, H_DIM),
                      Wl2, bl2.reshape(1, EMB_DIM))
    out_g = _tc_global(x, agg, deg,
                       Wg1[:D_FEAT], Wg1[D_FEAT:], bg1.reshape(1, H_DIM),
                       Wg2, bg2.reshape(1, EMB_DIM))
    return (out_g, out_l)


# speedup vs baseline: 13.0897x; 1.0038x over previous
import functools

import jax
import jax.numpy as jnp
from jax import lax
from jax.experimental import pallas as pl
from jax.experimental.pallas import tpu as pltpu
from jax.experimental.pallas import tpu_sc as plsc

N_NODES = 10000
N_EDGES = 320000
D_FEAT = 128
H_DIM = 512
EMB_DIM = 128

DW = 8
NC, NS = 2, 16
NW = NC * NS
EPW = N_EDGES // NW
CHUNK = 80
NCHUNK = EPW // CHUNK
G = 25
NGROUP = NCHUNK // G
RPT = 632
N_PAD = RPT * NS


def _sc_agg_body(x_hbm, er_hbm, zeros_hbm, ones_hbm,
                 agg_out, deg_out,
                 src_i, dst_i, ones_v, bufs, acc_sh, deg_sh,
                 gsems, ssems, dsem):
    cid = lax.axis_index("c")
    sid = lax.axis_index("s")
    wid = sid * NC + cid
    r0 = sid * RPT
    pltpu.sync_copy(ones_hbm, ones_v)
    pltpu.sync_copy(zeros_hbm, acc_sh.at[pl.ds(r0, RPT)])
    pltpu.sync_copy(zeros_hbm.at[pl.ds(0, RPT), pl.ds(0, DW)],
                    deg_sh.at[pl.ds(r0, RPT)])
    plsc.subcore_barrier()

    def gather(c, bi):
        pltpu.async_copy(x_hbm.at[src_i.at[c]], bufs[bi], gsems[bi])

    def gather_wait(c, bi):
        pltpu.make_async_copy(x_hbm.at[src_i.at[c]], bufs[bi],
                              gsems[bi]).wait()

    def scat(c, bi):
        pltpu.async_copy(bufs[bi], acc_sh.at[dst_i.at[c]], ssems[bi],
                         add=True)
        pltpu.async_copy(ones_v, deg_sh.at[dst_i.at[c]], dsem, add=True)

    def scat_wait(c, bi):
        pltpu.make_async_copy(bufs[bi], acc_sh.at[dst_i.at[c]],
                              ssems[bi]).wait()

    def deg_wait(c):
        pltpu.make_async_copy(ones_v, deg_sh.at[dst_i.at[c]], dsem).wait()

    def group(g, _):
        pltpu.sync_copy(er_hbm.at[0, wid, pl.ds(g * G, G)], src_i)
        pltpu.sync_copy(er_hbm.at[1, wid, pl.ds(g * G, G)], dst_i)
        gather(0, 0)
        gather(1, 1)
        for c in range(G):
            bi = c % 3
            gather_wait(c, bi)
            scat(c, bi)
            if c >= 1:
                deg_wait(c - 1)
            if c + 2 < G:
                nbi = (c + 2) % 3
                if c >= 1:
                    scat_wait(c - 1, nbi)
                gather(c + 2, nbi)
        for c in range(G - 3, G):
            scat_wait(c, c % 3)
        deg_wait(G - 1)
        return 0

    lax.fori_loop(0, NGROUP, group, 0)
    plsc.subcore_barrier()
    pltpu.sync_copy(acc_sh.at[pl.ds(r0, RPT)],
                    agg_out.at[cid, pl.ds(r0, RPT)])
    pltpu.sync_copy(deg_sh.at[pl.ds(r0, RPT)],
                    deg_out.at[cid, pl.ds(r0, RPT)])


_sc_agg = functools.partial(
    pl.kernel,
    out_type=(jax.ShapeDtypeStruct((NC, N_PAD, D_FEAT), jnp.float32),
              jax.ShapeDtypeStruct((NC, N_PAD, DW), jnp.float32)),
    mesh=plsc.VectorSubcoreMesh(core_axis_name="c", subcore_axis_name="s"),
    compiler_params=pltpu.CompilerParams(use_tc_tiling_on_sc=False),
    scratch_types=[
        pltpu.VMEM((G, CHUNK), jnp.int32),
        pltpu.VMEM((G, CHUNK), jnp.int32),
        pltpu.VMEM((CHUNK, DW), jnp.float32),
        [pltpu.VMEM((CHUNK, D_FEAT), jnp.float32)] * 3,
        pltpu.VMEM_SHARED((N_PAD, D_FEAT), jnp.float32),
        pltpu.VMEM_SHARED((N_PAD, DW), jnp.float32),
        [pltpu.SemaphoreType.DMA] * 3,
        [pltpu.SemaphoreType.DMA] * 3,
        pltpu.SemaphoreType.DMA,
    ],
)(_sc_agg_body)


BR = 1000


def _tc_global_body(x_ref, agg_ref, deg_ref, wg1_ref, bg1_ref,
                    wg2_ref, bg2_ref, outg_ref):
    xb = x_ref[...]
    a = agg_ref[0] + agg_ref[1]
    deg = deg_ref[0, :, 0:1] + deg_ref[1, :, 0:1]
    mean = a / jnp.maximum(deg, 1.0)
    h = jnp.maximum(
        jnp.dot(xb, wg1_ref[:D_FEAT], preferred_element_type=jnp.float32)
        + jnp.dot(mean, wg1_ref[D_FEAT:], preferred_element_type=jnp.float32)
        + bg1_ref[...], 0.0)
    outg_ref[...] = jnp.dot(h, wg2_ref[...],
                            preferred_element_type=jnp.float32) + bg2_ref[...]


def _tc_local_body(x_ref, wl1_ref, bl1_ref, wl2_ref, bl2_ref, outl_ref):
    hl = jnp.maximum(
        jnp.dot(x_ref[...], wl1_ref[...], preferred_element_type=jnp.float32)
        + bl1_ref[...], 0.0)
    outl_ref[...] = jnp.dot(hl, wl2_ref[...],
                            preferred_element_type=jnp.float32) + bl2_ref[...]


def _full(shape):
    return pl.BlockSpec(shape, lambda i: tuple(0 for _ in shape))


_tc_global = pl.pallas_call(
    _tc_global_body,
    grid=(N_NODES // BR,),
    in_specs=[
        pl.BlockSpec((BR, D_FEAT), lambda i: (i, 0)),
        pl.BlockSpec((NC, BR, D_FEAT), lambda i: (0, i, 0)),
        pl.BlockSpec((NC, BR, DW), lambda i: (0, i, 0)),
        _full((2 * D_FEAT, H_DIM)),
        _full((H_DIM,)),
        _full((H_DIM, EMB_DIM)),
        _full((EMB_DIM,)),
    ],
    out_specs=pl.BlockSpec((BR, EMB_DIM), lambda i: (i, 0)),
    out_shape=jax.ShapeDtypeStruct((N_NODES, EMB_DIM), jnp.float32),
)

_tc_local = pl.pallas_call(
    _tc_local_body,
    grid=(N_NODES // BR,),
    in_specs=[
        pl.BlockSpec((BR, D_FEAT), lambda i: (i, 0)),
        _full((D_FEAT, H_DIM)),
        _full((H_DIM,)),
        _full((H_DIM, EMB_DIM)),
        _full((EMB_DIM,)),
    ],
    out_specs=pl.BlockSpec((BR, EMB_DIM), lambda i: (i, 0)),
    out_shape=jax.ShapeDtypeStruct((N_NODES, EMB_DIM), jnp.float32),
)


def kernel(x, edge_index, Wg1, bg1, Wg2, bg2, Wl1, bl1, Wl2, bl2):
    er = edge_index.reshape(2, NW, NCHUNK, CHUNK)
    zeros = jnp.zeros((RPT, D_FEAT), jnp.float32)
    ones = jnp.ones((CHUNK, DW), jnp.float32)
    agg, deg = _sc_agg(x, er, zeros, ones)
    out_l = _tc_local(x, Wl1, bl1, Wl2, bl2)
    out_g = _tc_global(x, agg, deg, Wg1, bg1, Wg2, bg2)
    return (out_g, out_l)


# speedup vs baseline: 13.0937x; 1.0003x over previous
import functools

import jax
import jax.numpy as jnp
from jax import lax
from jax.experimental import pallas as pl
from jax.experimental.pallas import tpu as pltpu
from jax.experimental.pallas import tpu_sc as plsc

N_NODES = 10000
N_EDGES = 320000
D_FEAT = 128
H_DIM = 512
EMB_DIM = 128

DW = 8
NC, NS = 2, 16
NW = NC * NS
EPW = N_EDGES // NW
CHUNK = 80
NCHUNK = EPW // CHUNK
G = 25
NGROUP = NCHUNK // G
RPT = 632
N_PAD = RPT * NS


def _sc_agg_body(x_hbm, er_hbm, zeros_hbm, ones_hbm,
                 agg_out, deg_out,
                 src_i, dst_i, ones_v, bufs, acc_sh, deg_sh,
                 gsems, ssems, dsem):
    cid = lax.axis_index("c")
    sid = lax.axis_index("s")
    wid = sid * NC + cid
    r0 = sid * RPT
    pltpu.sync_copy(ones_hbm, ones_v)
    pltpu.sync_copy(zeros_hbm, acc_sh.at[pl.ds(r0, RPT)])
    pltpu.sync_copy(zeros_hbm.at[pl.ds(0, RPT), pl.ds(0, DW)],
                    deg_sh.at[pl.ds(r0, RPT)])
    plsc.subcore_barrier()

    def gather(c, bi):
        pltpu.async_copy(x_hbm.at[src_i.at[c]], bufs[bi], gsems[bi])

    def gather_wait(c, bi):
        pltpu.make_async_copy(x_hbm.at[src_i.at[c]], bufs[bi],
                              gsems[bi]).wait()

    def scat(c, bi):
        pltpu.async_copy(bufs[bi], acc_sh.at[dst_i.at[c]], ssems[bi],
                         add=True)
        pltpu.async_copy(ones_v, deg_sh.at[dst_i.at[c]], dsem, add=True)

    def scat_wait(c, bi):
        pltpu.make_async_copy(bufs[bi], acc_sh.at[dst_i.at[c]],
                              ssems[bi]).wait()

    def deg_wait(c):
        pltpu.make_async_copy(ones_v, deg_sh.at[dst_i.at[c]], dsem).wait()

    def group(g, _):
        pltpu.sync_copy(er_hbm.at[0, wid, pl.ds(g * G, G)], src_i)
        pltpu.sync_copy(er_hbm.at[1, wid, pl.ds(g * G, G)], dst_i)
        gather(0, 0)
        gather(1, 1)
        for c in range(G):
            bi = c % 3
            gather_wait(c, bi)
            scat(c, bi)
            if c >= 1:
                deg_wait(c - 1)
            if c + 2 < G:
                nbi = (c + 2) % 3
                if c >= 1:
                    scat_wait(c - 1, nbi)
                gather(c + 2, nbi)
        for c in range(G - 3, G):
            scat_wait(c, c % 3)
        deg_wait(G - 1)
        return 0

    lax.fori_loop(0, NGROUP, group, 0)
    plsc.subcore_barrier()
    pltpu.sync_copy(acc_sh.at[pl.ds(r0, RPT)],
                    agg_out.at[cid, pl.ds(r0, RPT)])
    pltpu.sync_copy(deg_sh.at[pl.ds(r0, RPT)],
                    deg_out.at[cid, pl.ds(r0, RPT)])


_sc_agg = functools.partial(
    pl.kernel,
    out_type=(jax.ShapeDtypeStruct((NC, N_PAD, D_FEAT), jnp.float32),
              jax.ShapeDtypeStruct((NC, N_PAD, DW), jnp.float32)),
    mesh=plsc.VectorSubcoreMesh(core_axis_name="c", subcore_axis_name="s"),
    compiler_params=pltpu.CompilerParams(use_tc_tiling_on_sc=False),
    scratch_types=[
        pltpu.VMEM((G, CHUNK), jnp.int32),
        pltpu.VMEM((G, CHUNK), jnp.int32),
        pltpu.VMEM((CHUNK, DW), jnp.float32),
        [pltpu.VMEM((CHUNK, D_FEAT), jnp.float32)] * 3,
        pltpu.VMEM_SHARED((N_PAD, D_FEAT), jnp.float32),
        pltpu.VMEM_SHARED((N_PAD, DW), jnp.float32),
        [pltpu.SemaphoreType.DMA] * 3,
        [pltpu.SemaphoreType.DMA] * 3,
        pltpu.SemaphoreType.DMA,
    ],
)(_sc_agg_body)


BR = 1000


def _tc_global_body(x_ref, agg_ref, deg_ref, wg1_ref, bg1_ref,
                    wg2_ref, bg2_ref, outg_ref):
    xb = x_ref[...]
    a = agg_ref[0] + agg_ref[1]
    deg = deg_ref[0, :, 0:1] + deg_ref[1, :, 0:1]
    mean = a / jnp.maximum(deg, 1.0)
    h = jnp.maximum(
        jnp.dot(xb, wg1_ref[:D_FEAT], preferred_element_type=jnp.float32)
        + jnp.dot(mean, wg1_ref[D_FEAT:], preferred_element_type=jnp.float32)
        + bg1_ref[...], 0.0)
    outg_ref[...] = jnp.dot(h, wg2_ref[...],
                            preferred_element_type=jnp.float32) + bg2_ref[...]


def _tc_local_body(x_ref, wl1_ref, bl1_ref, wl2_ref, bl2_ref, outl_ref):
    hl = jnp.maximum(
        jnp.dot(x_ref[...], wl1_ref[...], preferred_element_type=jnp.float32)
        + bl1_ref[...], 0.0)
    outl_ref[...] = jnp.dot(hl, wl2_ref[...],
                            preferred_element_type=jnp.float32) + bl2_ref[...]


def _full(shape):
    return pl.BlockSpec(shape, lambda i: tuple(0 for _ in shape))


_tc_global = pl.pallas_call(
    _tc_global_body,
    grid=(N_NODES // BR,),
    in_specs=[
        pl.BlockSpec((BR, D_FEAT), lambda i: (i, 0)),
        pl.BlockSpec((NC, BR, D_FEAT), lambda i: (0, i, 0)),
        pl.BlockSpec((NC, BR, DW), lambda i: (0, i, 0)),
        _full((2 * D_FEAT, H_DIM)),
        _full((H_DIM,)),
        _full((H_DIM, EMB_DIM)),
        _full((EMB_DIM,)),
    ],
    out_specs=pl.BlockSpec((BR, EMB_DIM), lambda i: (i, 0)),
    out_shape=jax.ShapeDtypeStruct((N_NODES, EMB_DIM), jnp.float32),
)

_tc_local = pl.pallas_call(
    _tc_local_body,
    grid=(N_NODES // BR,),
    in_specs=[
        pl.BlockSpec((BR, D_FEAT), lambda i: (i, 0)),
        _full((D_FEAT, H_DIM)),
        _full((H_DIM,)),
        _full((H_DIM, EMB_DIM)),
        _full((EMB_DIM,)),
    ],
    out_specs=pl.BlockSpec((BR, EMB_DIM), lambda i: (i, 0)),
    out_shape=jax.ShapeDtypeStruct((N_NODES, EMB_DIM), jnp.float32),
)


def kernel(x, edge_index, Wg1, bg1, Wg2, bg2, Wl1, bl1, Wl2, bl2):
    er = edge_index.reshape(2, NW, NCHUNK, CHUNK)
    zeros = jnp.zeros((RPT, D_FEAT), jnp.float32)
    ones = jnp.ones((CHUNK, DW), jnp.float32)
    out_l = _tc_local(x, Wl1, bl1, Wl2, bl2)
    agg, deg = _sc_agg(x, er, zeros, ones)
    out_g = _tc_global(x, agg, deg, Wg1, bg1, Wg2, bg2)
    return (out_g, out_l)
